# baseline (speedup 1.0000x reference)
import jax
import jax.numpy as jnp
from jax import lax
from jax.experimental import pallas as pl
from jax.experimental.pallas import tpu as pltpu
from jax.experimental.pallas import tpu_sc as plsc

_N = 10000
_E = 160000
_D = 256
_H = 128
_EPS = 1e-6
_SELU_ALPHA = 1.6732632423543772
_SELU_SCALE = 1.0507009873554805


_R = 1000
_TC_GRID = _N // _R


def _spatial_mask(r):
    return lax.broadcasted_iota(jnp.int32, (r, _D), 1) >= 1


def _sumsq_sp(v, m):
    vm = jnp.where(m, v, 0.0)
    return jnp.sum(vm * vm, axis=1, keepdims=True)


def _g_from_x(xf, m):
    kf = xf / xf[:, 0:1]
    n2 = jnp.clip(_sumsq_sp(kf, m), 0.0, 0.9)
    lamb = 1.0 / jnp.sqrt(1.0 - n2)
    return lamb * kf


def _post_agg(a, m):
    den = a[:, 0:1]
    safe = jnp.where(den == 0.0, 1.0, den)
    kf = a / safe
    e0 = jnp.where(m, 0.0, 1.0)
    kf = jnp.where(den == 0.0, e0, kf)
    n2m = _sumsq_sp(kf, m)
    x0 = 1.0 / jnp.sqrt(jnp.clip(1.0 - n2m, _EPS, None))
    pf = jnp.where(m, (x0 * kf) / (x0 + 1.0), 0.0)
    sel = _SELU_SCALE * jnp.where(pf > 0.0, pf,
                                  _SELU_ALPHA * (jnp.exp(pf) - 1.0))
    n2p = _sumsq_sp(sel, m)
    denom = jnp.clip(1.0 - n2p, _EPS, None)
    xsb = 2.0 * sel / denom
    n2s = _sumsq_sp(xsb, m)
    x0c = jnp.sqrt(1.0 + n2s)
    return jnp.where(m, xsb, x0c)


def _first_body(x_ref, g_ref):
    x = x_ref[...]
    m = _spatial_mask(_R)
    g = _g_from_x(x, m)
    g_ref[0] = g[:, :_H]
    g_ref[1] = g[:, _H:]


def _mid_body(a_ref, g_ref):
    a = jnp.concatenate([a_ref[0], a_ref[1]], axis=1)
    m = _spatial_mask(_R)
    g = _g_from_x(_post_agg(a, m), m)
    g_ref[0] = g[:, :_H]
    g_ref[1] = g[:, _H:]


def _last_body(a_ref, x_ref):
    a = jnp.concatenate([a_ref[0], a_ref[1]], axis=1)
    m = _spatial_mask(_R)
    x_ref[...] = _post_agg(a, m)


_g_spec = pl.BlockSpec((2, _R, _H), lambda i: (0, i, 0))
_x_spec = pl.BlockSpec((_R, _D), lambda i: (i, 0))

_node_first = pl.pallas_call(
    _first_body, grid=(_TC_GRID,), in_specs=[_x_spec], out_specs=_g_spec,
    out_shape=jax.ShapeDtypeStruct((2, _N, _H), jnp.float32))

_node_mid = pl.pallas_call(
    _mid_body, grid=(_TC_GRID,), in_specs=[_g_spec], out_specs=_g_spec,
    out_shape=jax.ShapeDtypeStruct((2, _N, _H), jnp.float32))

_node_last = pl.pallas_call(
    _last_body, grid=(_TC_GRID,), in_specs=[_g_spec], out_specs=_x_spec,
    out_shape=jax.ShapeDtypeStruct((_N, _D), jnp.float32))


_NS = 16
_EPT = _E // _NS
_C = 128
_NF = _EPT // _C
_CT = _EPT - _NF * _C
_RB = 624
_ZR = 208
_NZ = _RB // _ZR
_TAIL = _N - _RB * _NS


_GATHER_DN = lax.GatherDimensionNumbers(
    offset_dims=(), collapsed_slice_dims=(0,), start_index_map=(0,))


def _bcast_lane(wv, l):
    idx = jnp.full((16, 1), l, jnp.int32)
    return lax.gather(wv, idx, _GATHER_DN, slice_sizes=(1,),
                      mode=lax.GatherScatterMode.PROMISE_IN_BOUNDS)


def _sc_body(g_hbm, colp_hbm, row_hbm, w_hbm, out_hbm,
             colbig, rowbuf0, rowbuf1, wbuf0, wbuf1, rowsv0, rowsv1,
             rowbuft, wbuft, acc_sh, semg, semi):
    c = lax.axis_index("c")
    s = lax.axis_index("s")
    ebase = s * _EPT

    pltpu.sync_copy(colp_hbm.at[pl.ds(c * _E + ebase, _EPT)], colbig)

    def _zrow(i, carry):
        for l in range(_H // 16):
            rowsv0[i, pl.ds(l * 16, 16)] = jnp.zeros((16,), jnp.float32)
        return carry
    lax.fori_loop(0, _C, _zrow, 0)
    rbase = s * _RB
    for b in range(_RB // _C):
        pltpu.sync_copy(rowsv0, acc_sh.at[pl.ds(rbase + b * _C, _C)])
    ztail = _RB - (_RB // _C) * _C
    pltpu.sync_copy(rowsv0.at[pl.ds(0, ztail)],
                    acc_sh.at[pl.ds(rbase + _RB - ztail, ztail)])

    @pl.when(s == 0)
    def _zero_tail():
        pltpu.sync_copy(rowsv0.at[pl.ds(0, _TAIL)],
                        acc_sh.at[pl.ds(_RB * _NS, _TAIL)])
    plsc.subcore_barrier()

    def _issue(j_off, n, rowbuf, wbuf, rowsv):
        pltpu.async_copy(row_hbm.at[pl.ds(ebase + j_off, n)], rowbuf, semi)
        pltpu.async_copy(w_hbm.at[pl.ds(ebase + j_off, n)], wbuf, semi)
        pltpu.async_copy(g_hbm.at[colbig.at[pl.ds(j_off, n)]], rowsv, semg)

    def _wait(rowbuf, wbuf, rowsv):
        pltpu.make_async_copy(row_hbm.at[pl.ds(0, _C)], rowbuf, semi).wait()
        pltpu.make_async_copy(w_hbm.at[pl.ds(0, _C)], wbuf, semi).wait()
        pltpu.make_async_copy(g_hbm.at[colbig.at[pl.ds(0, _C)]],
                              rowsv, semg).wait()

    def _scale(n, wbuf, rowsv):
        for j in range(n // 16):
            wv = wbuf[pl.ds(j * 16, 16)]
            for l in range(16):
                wb = _bcast_lane(wv, l)
                e = j * 16 + l
                for q in range(_H // 16):
                    sl = pl.ds(q * 16, 16)
                    rowsv[e, sl] = rowsv[e, sl] * wb

    _issue(0, _C, rowbuf0, wbuf0, rowsv0)
    _issue(_C, _C, rowbuf1, wbuf1, rowsv1)

    def _pair(i, carry):
        off = i * 2 * _C
        _wait(rowbuf0, wbuf0, rowsv0)
        _scale(_C, wbuf0, rowsv0)
        pltpu.sync_copy(rowsv0, acc_sh.at[rowbuf0], add=True)

        @pl.when(i < _NF // 2 - 1)
        def _refill0():
            _issue(off + 2 * _C, _C, rowbuf0, wbuf0, rowsv0)

        @pl.when(i == _NF // 2 - 1)
        def _refill0t():
            _issue(_NF * _C, _CT, rowbuft, wbuft,
                   rowsv0.at[pl.ds(0, _CT)])
        _wait(rowbuf1, wbuf1, rowsv1)
        _scale(_C, wbuf1, rowsv1)
        pltpu.sync_copy(rowsv1, acc_sh.at[rowbuf1], add=True)

        @pl.when(i < _NF // 2 - 1)
        def _refill1():
            _issue(off + 3 * _C, _C, rowbuf1, wbuf1, rowsv1)
        return carry
    lax.fori_loop(0, _NF // 2, _pair, 0)

    pltpu.make_async_copy(row_hbm.at[pl.ds(0, _CT)], rowbuft, semi).wait()
    pltpu.make_async_copy(w_hbm.at[pl.ds(0, _CT)], wbuft, semi).wait()
    pltpu.make_async_copy(g_hbm.at[colbig.at[pl.ds(0, _CT)]],
                          rowsv0.at[pl.ds(0, _CT)], semg).wait()
    _scale(_CT, wbuft, rowsv0)
    pltpu.sync_copy(rowsv0.at[pl.ds(0, _CT)], acc_sh.at[rowbuft], add=True)

    plsc.subcore_barrier()
    for b in range(_NZ):
        sl = pl.ds(rbase + b * _ZR, _ZR)
        pltpu.sync_copy(acc_sh.at[sl], out_hbm.at[c, sl])

    @pl.when(s == 0)
    def _copy_tail():
        sl = pl.ds(_RB * _NS, _TAIL)
        pltpu.sync_copy(acc_sh.at[sl], out_hbm.at[c, sl])


_sc_agg_cached = None


def _sc_agg(*args):
    global _sc_agg_cached
    if _sc_agg_cached is None:
        mesh = plsc.VectorSubcoreMesh(core_axis_name="c", subcore_axis_name="s")
        _sc_agg_cached = pl.kernel(
            _sc_body, mesh=mesh,
            out_type=jax.ShapeDtypeStruct((2, _N, _H), jnp.float32),
            scratch_types=[
                pltpu.VMEM((_EPT,), jnp.int32),
                pltpu.VMEM((_C,), jnp.int32),
                pltpu.VMEM((_C,), jnp.int32),
                pltpu.VMEM((_C,), jnp.float32),
                pltpu.VMEM((_C,), jnp.float32),
                pltpu.VMEM((_C, _H), jnp.float32),
                pltpu.VMEM((_C, _H), jnp.float32),
                pltpu.VMEM((_CT,), jnp.int32),
                pltpu.VMEM((_CT,), jnp.float32),
                pltpu.VMEM_SHARED((_N, _H), jnp.float32),
                pltpu.SemaphoreType.DMA,
                pltpu.SemaphoreType.DMA,
            ])
    return _sc_agg_cached(*args)


def kernel(node_repr, edge_index, edge_weight):
    row = edge_index[0].astype(jnp.int32)
    col = edge_index[1].astype(jnp.int32)
    colp = jnp.concatenate([col, col + _N])
    w = edge_weight.astype(jnp.float32)

    g = _node_first(node_repr.astype(jnp.float32))
    a = _sc_agg(g.reshape(2 * _N, _H), colp, row, w)
    g = _node_mid(a)
    a = _sc_agg(g.reshape(2 * _N, _H), colp, row, w)
    return _node_last(a)


# speedup vs baseline: 1.1870x; 1.1870x over previous
import jax
import jax.numpy as jnp
from jax import lax
from jax.experimental import pallas as pl
from jax.experimental.pallas import tpu as pltpu
from jax.experimental.pallas import tpu_sc as plsc

_N = 10000
_E = 160000
_D = 256
_H = 128
_EPS = 1e-6
_SELU_ALPHA = 1.6732632423543772
_SELU_SCALE = 1.0507009873554805


_R = 1000
_TC_GRID = _N // _R


def _spatial_mask(r):
    return lax.broadcasted_iota(jnp.int32, (r, _D), 1) >= 1


def _sumsq_sp(v, m):
    vm = jnp.where(m, v, 0.0)
    return jnp.sum(vm * vm, axis=1, keepdims=True)


def _g_from_x(xf, m):
    kf = xf / xf[:, 0:1]
    n2 = jnp.clip(_sumsq_sp(kf, m), 0.0, 0.9)
    lamb = 1.0 / jnp.sqrt(1.0 - n2)
    return lamb * kf


def _post_agg(a, m):
    den = a[:, 0:1]
    safe = jnp.where(den == 0.0, 1.0, den)
    kf = a / safe
    e0 = jnp.where(m, 0.0, 1.0)
    kf = jnp.where(den == 0.0, e0, kf)
    n2m = _sumsq_sp(kf, m)
    x0 = 1.0 / jnp.sqrt(jnp.clip(1.0 - n2m, _EPS, None))
    pf = jnp.where(m, (x0 * kf) / (x0 + 1.0), 0.0)
    sel = _SELU_SCALE * jnp.where(pf > 0.0, pf,
                                  _SELU_ALPHA * (jnp.exp(pf) - 1.0))
    n2p = _sumsq_sp(sel, m)
    denom = jnp.clip(1.0 - n2p, _EPS, None)
    xsb = 2.0 * sel / denom
    n2s = _sumsq_sp(xsb, m)
    x0c = jnp.sqrt(1.0 + n2s)
    return jnp.where(m, xsb, x0c)


def _first_body(x_ref, g_ref):
    x = x_ref[...]
    m = _spatial_mask(_R)
    g = _g_from_x(x, m)
    g_ref[0] = g[:, :_H]
    g_ref[1] = g[:, _H:]


def _mid_body(a_ref, g_ref):
    a = jnp.concatenate([a_ref[0], a_ref[1]], axis=1)
    m = _spatial_mask(_R)
    g = _g_from_x(_post_agg(a, m), m)
    g_ref[0] = g[:, :_H]
    g_ref[1] = g[:, _H:]


def _last_body(a_ref, x_ref):
    a = jnp.concatenate([a_ref[0], a_ref[1]], axis=1)
    m = _spatial_mask(_R)
    x_ref[...] = _post_agg(a, m)


_g_spec = pl.BlockSpec((2, _R, _H), lambda i: (0, i, 0))
_x_spec = pl.BlockSpec((_R, _D), lambda i: (i, 0))

_node_first = pl.pallas_call(
    _first_body, grid=(_TC_GRID,), in_specs=[_x_spec], out_specs=_g_spec,
    out_shape=jax.ShapeDtypeStruct((2, _N, _H), jnp.float32))

_node_mid = pl.pallas_call(
    _mid_body, grid=(_TC_GRID,), in_specs=[_g_spec], out_specs=_g_spec,
    out_shape=jax.ShapeDtypeStruct((2, _N, _H), jnp.float32))

_node_last = pl.pallas_call(
    _last_body, grid=(_TC_GRID,), in_specs=[_g_spec], out_specs=_x_spec,
    out_shape=jax.ShapeDtypeStruct((_N, _D), jnp.float32))


_NS = 16
_EPT = _E // _NS
_C = 80
_NCH = _EPT // _C
_NT = 41
_RB = 624
_ZR = 208
_NZ = _RB // _ZR
_TAIL = _N - _RB * _NS


_GATHER_DN = lax.GatherDimensionNumbers(
    offset_dims=(), collapsed_slice_dims=(0,), start_index_map=(0,))


def _bcast_lane(wv, l):
    idx = jnp.full((16, 1), l, jnp.int32)
    return lax.gather(wv, idx, _GATHER_DN, slice_sizes=(1,),
                      mode=lax.GatherScatterMode.PROMISE_IN_BOUNDS)


def _sc_body(g_hbm, colp_hbm, row_hbm, w_hbm, out_hbm,
             colbig, rowbuf0, rowbuf1, rowbuf2, wbuf0, wbuf1, wbuf2,
             rowsv0, rowsv1, rowsv2, acc_sh, semg, semi, sems):
    c = lax.axis_index("c")
    s = lax.axis_index("s")
    ebase = s * _EPT
    bufs = ((rowbuf0, wbuf0, rowsv0),
            (rowbuf1, wbuf1, rowsv1),
            (rowbuf2, wbuf2, rowsv2))

    pltpu.sync_copy(colp_hbm.at[pl.ds(c * _E + ebase, _EPT)], colbig)

    def _zrow(i, carry):
        for l in range(_H // 16):
            rowsv0[i, pl.ds(l * 16, 16)] = jnp.zeros((16,), jnp.float32)
        return carry
    lax.fori_loop(0, _C, _zrow, 0)
    rbase = s * _RB
    for b in range(_RB // _C):
        pltpu.sync_copy(rowsv0, acc_sh.at[pl.ds(rbase + b * _C, _C)])
    ztail = _RB - (_RB // _C) * _C
    pltpu.sync_copy(rowsv0.at[pl.ds(0, ztail)],
                    acc_sh.at[pl.ds(rbase + _RB - ztail, ztail)])

    @pl.when(s == 0)
    def _zero_tail():
        pltpu.sync_copy(rowsv0.at[pl.ds(0, _TAIL)],
                        acc_sh.at[pl.ds(_RB * _NS, _TAIL)])
    plsc.subcore_barrier()

    def _issue(j_off, bufset):
        rowbuf, wbuf, rowsv = bufset
        pltpu.async_copy(row_hbm.at[pl.ds(ebase + j_off, _C)], rowbuf, semi)
        pltpu.async_copy(w_hbm.at[pl.ds(ebase + j_off, _C)], wbuf, semi)
        pltpu.async_copy(g_hbm.at[colbig.at[pl.ds(j_off, _C)]], rowsv, semg)

    def _wait_in(bufset):
        rowbuf, wbuf, rowsv = bufset
        pltpu.make_async_copy(row_hbm.at[pl.ds(0, _C)], rowbuf, semi).wait()
        pltpu.make_async_copy(w_hbm.at[pl.ds(0, _C)], wbuf, semi).wait()
        pltpu.make_async_copy(g_hbm.at[colbig.at[pl.ds(0, _C)]],
                              rowsv, semg).wait()

    def _scale(bufset):
        _, wbuf, rowsv = bufset
        for j in range(_C // 16):
            wv = wbuf[pl.ds(j * 16, 16)]
            for l in range(16):
                wb = _bcast_lane(wv, l)
                e = j * 16 + l
                for q in range(_H // 16):
                    sl = pl.ds(q * 16, 16)
                    rowsv[e, sl] = rowsv[e, sl] * wb

    def _scatter_start(bufset):
        rowbuf, _, rowsv = bufset
        pltpu.async_copy(rowsv, acc_sh.at[rowbuf], sems, add=True)

    def _scatter_wait(bufset):
        pltpu.make_async_copy(g_hbm.at[pl.ds(0, _C)], bufset[2], sems).wait()

    _issue(0, bufs[0])
    _issue(_C, bufs[1])

    def _ring(i, carry):
        for k in range(3):
            p, q = k, (k + 2) % 3
            _wait_in(bufs[p])
            _scale(bufs[p])
            _scatter_start(bufs[p])
            if k == 0:
                @pl.when(i >= 1)
                def _sw():
                    _scatter_wait(bufs[q])
            else:
                _scatter_wait(bufs[q])
            _issue((i * 3 + k + 2) * _C, bufs[q])
        return carry
    lax.fori_loop(0, _NT, _ring, 0)

    _wait_in(bufs[0])
    _scale(bufs[0])
    _scatter_start(bufs[0])
    _wait_in(bufs[1])
    _scale(bufs[1])
    _scatter_start(bufs[1])
    _scatter_wait(bufs[2])
    _scatter_wait(bufs[0])
    _scatter_wait(bufs[1])

    plsc.subcore_barrier()
    for b in range(_NZ):
        sl = pl.ds(rbase + b * _ZR, _ZR)
        pltpu.sync_copy(acc_sh.at[sl], out_hbm.at[c, sl])

    @pl.when(s == 0)
    def _copy_tail():
        sl = pl.ds(_RB * _NS, _TAIL)
        pltpu.sync_copy(acc_sh.at[sl], out_hbm.at[c, sl])


_sc_agg_cached = None


def _sc_agg(*args):
    global _sc_agg_cached
    if _sc_agg_cached is None:
        mesh = plsc.VectorSubcoreMesh(core_axis_name="c", subcore_axis_name="s")
        _sc_agg_cached = pl.kernel(
            _sc_body, mesh=mesh,
            out_type=jax.ShapeDtypeStruct((2, _N, _H), jnp.float32),
            scratch_types=[
                pltpu.VMEM((_EPT,), jnp.int32),
                pltpu.VMEM((_C,), jnp.int32),
                pltpu.VMEM((_C,), jnp.int32),
                pltpu.VMEM((_C,), jnp.int32),
                pltpu.VMEM((_C,), jnp.float32),
                pltpu.VMEM((_C,), jnp.float32),
                pltpu.VMEM((_C,), jnp.float32),
                pltpu.VMEM((_C, _H), jnp.float32),
                pltpu.VMEM((_C, _H), jnp.float32),
                pltpu.VMEM((_C, _H), jnp.float32),
                pltpu.VMEM_SHARED((_N, _H), jnp.float32),
                pltpu.SemaphoreType.DMA,
                pltpu.SemaphoreType.DMA,
                pltpu.SemaphoreType.DMA,
            ])
    return _sc_agg_cached(*args)


def kernel(node_repr, edge_index, edge_weight):
    row = edge_index[0].astype(jnp.int32)
    col = edge_index[1].astype(jnp.int32)
    colp = jnp.concatenate([col, col + _N])
    w = edge_weight.astype(jnp.float32)

    g = _node_first(node_repr.astype(jnp.float32))
    a = _sc_agg(g.reshape(2 * _N, _H), colp, row, w)
    g = _node_mid(a)
    a = _sc_agg(g.reshape(2 * _N, _H), colp, row, w)
    return _node_last(a)


# speedup vs baseline: 1.2027x; 1.0133x over previous
import jax
import jax.numpy as jnp
from jax import lax
from jax.experimental import pallas as pl
from jax.experimental.pallas import tpu as pltpu
from jax.experimental.pallas import tpu_sc as plsc

_N = 10000
_E = 160000
_D = 256
_H = 128
_EPS = 1e-6
_SELU_ALPHA = 1.6732632423543772
_SELU_SCALE = 1.0507009873554805


_R = 1000
_TC_GRID = _N // _R


def _spatial_mask(r):
    return lax.broadcasted_iota(jnp.int32, (r, _D), 1) >= 1


def _sumsq_sp(v, m):
    vm = jnp.where(m, v, 0.0)
    return jnp.sum(vm * vm, axis=1, keepdims=True)


def _g_from_x(xf, m):
    kf = xf / xf[:, 0:1]
    n2 = jnp.clip(_sumsq_sp(kf, m), 0.0, 0.9)
    lamb = 1.0 / jnp.sqrt(1.0 - n2)
    return lamb * kf


def _post_agg(a, m):
    den = a[:, 0:1]
    safe = jnp.where(den == 0.0, 1.0, den)
    kf = a / safe
    e0 = jnp.where(m, 0.0, 1.0)
    kf = jnp.where(den == 0.0, e0, kf)
    n2m = _sumsq_sp(kf, m)
    x0 = 1.0 / jnp.sqrt(jnp.clip(1.0 - n2m, _EPS, None))
    pf = jnp.where(m, (x0 * kf) / (x0 + 1.0), 0.0)
    sel = _SELU_SCALE * jnp.where(pf > 0.0, pf,
                                  _SELU_ALPHA * (jnp.exp(pf) - 1.0))
    n2p = _sumsq_sp(sel, m)
    denom = jnp.clip(1.0 - n2p, _EPS, None)
    xsb = 2.0 * sel / denom
    n2s = _sumsq_sp(xsb, m)
    x0c = jnp.sqrt(1.0 + n2s)
    return jnp.where(m, xsb, x0c)


def _first_body(x_ref, g_ref):
    x = x_ref[...]
    m = _spatial_mask(_R)
    g = _g_from_x(x, m)
    g_ref[0] = g[:, :_H]
    g_ref[1] = g[:, _H:]


def _mid_body(a_ref, g_ref):
    a = jnp.concatenate([a_ref[0], a_ref[1]], axis=1)
    m = _spatial_mask(_R)
    g = _g_from_x(_post_agg(a, m), m)
    g_ref[0] = g[:, :_H]
    g_ref[1] = g[:, _H:]


def _last_body(a_ref, x_ref):
    a = jnp.concatenate([a_ref[0], a_ref[1]], axis=1)
    m = _spatial_mask(_R)
    x_ref[...] = _post_agg(a, m)


_g_spec = pl.BlockSpec((2, _R, _H), lambda i: (0, i, 0))
_x_spec = pl.BlockSpec((_R, _D), lambda i: (i, 0))

_node_first = pl.pallas_call(
    _first_body, grid=(_TC_GRID,), in_specs=[_x_spec], out_specs=_g_spec,
    out_shape=jax.ShapeDtypeStruct((2, _N, _H), jnp.float32))

_node_mid = pl.pallas_call(
    _mid_body, grid=(_TC_GRID,), in_specs=[_g_spec], out_specs=_g_spec,
    out_shape=jax.ShapeDtypeStruct((2, _N, _H), jnp.float32))

_node_last = pl.pallas_call(
    _last_body, grid=(_TC_GRID,), in_specs=[_g_spec], out_specs=_x_spec,
    out_shape=jax.ShapeDtypeStruct((_N, _D), jnp.float32))


_NS = 16
_EPT = _E // _NS
_C = 80
_NCH = _EPT // _C
_NT = 41
_RB = 624
_ZR = 208
_NZ = _RB // _ZR
_TAIL = _N - _RB * _NS


_GATHER_DN = lax.GatherDimensionNumbers(
    offset_dims=(), collapsed_slice_dims=(0,), start_index_map=(0,))


def _bcast_lane(wv, l):
    idx = jnp.full((16, 1), l, jnp.int32)
    return lax.gather(wv, idx, _GATHER_DN, slice_sizes=(1,),
                      mode=lax.GatherScatterMode.PROMISE_IN_BOUNDS)


def _sc_body(g_hbm, colp_hbm, row_hbm, w_hbm, out_hbm,
             colbig, rowbuf0, rowbuf1, rowbuf2, wbuf0, wbuf1, wbuf2,
             rowsv0, rowsv1, rowsv2, acc_sh, semg, semi, sems):
    c = lax.axis_index("c")
    s = lax.axis_index("s")
    ebase = s * _EPT
    bufs = ((rowbuf0, wbuf0, rowsv0),
            (rowbuf1, wbuf1, rowsv1),
            (rowbuf2, wbuf2, rowsv2))

    pltpu.async_copy(colp_hbm.at[pl.ds(c * _E + ebase, _EPT)], colbig, semg)

    def _zrow(i, carry):
        for l in range(_H // 16):
            rowsv0[i, pl.ds(l * 16, 16)] = jnp.zeros((16,), jnp.float32)
        return carry
    lax.fori_loop(0, _C, _zrow, 0)
    rbase = s * _RB
    ztail = _RB - (_RB // _C) * _C
    for b in range(_RB // _C):
        pltpu.async_copy(rowsv0, acc_sh.at[pl.ds(rbase + b * _C, _C)], semi)
    pltpu.async_copy(rowsv0.at[pl.ds(0, ztail)],
                     acc_sh.at[pl.ds(rbase + _RB - ztail, ztail)], semi)

    @pl.when(s == 0)
    def _zero_tail():
        pltpu.async_copy(rowsv0.at[pl.ds(0, _TAIL)],
                         acc_sh.at[pl.ds(_RB * _NS, _TAIL)], semi)
    for b in range(_RB // _C):
        pltpu.make_async_copy(rowsv0, acc_sh.at[pl.ds(rbase, _C)],
                              semi).wait()
    pltpu.make_async_copy(rowsv0.at[pl.ds(0, ztail)],
                          acc_sh.at[pl.ds(rbase, ztail)], semi).wait()

    @pl.when(s == 0)
    def _zero_tail_wait():
        pltpu.make_async_copy(rowsv0.at[pl.ds(0, _TAIL)],
                              acc_sh.at[pl.ds(rbase, _TAIL)], semi).wait()
    pltpu.make_async_copy(colp_hbm.at[pl.ds(0, _EPT)], colbig, semg).wait()
    plsc.subcore_barrier()

    def _issue(j_off, bufset):
        rowbuf, wbuf, rowsv = bufset
        pltpu.async_copy(row_hbm.at[pl.ds(ebase + j_off, _C)], rowbuf, semi)
        pltpu.async_copy(w_hbm.at[pl.ds(ebase + j_off, _C)], wbuf, semi)
        pltpu.async_copy(g_hbm.at[colbig.at[pl.ds(j_off, _C)]], rowsv, semg)

    def _wait_in(bufset):
        rowbuf, wbuf, rowsv = bufset
        pltpu.make_async_copy(row_hbm.at[pl.ds(0, _C)], rowbuf, semi).wait()
        pltpu.make_async_copy(w_hbm.at[pl.ds(0, _C)], wbuf, semi).wait()
        pltpu.make_async_copy(g_hbm.at[colbig.at[pl.ds(0, _C)]],
                              rowsv, semg).wait()

    def _scale(bufset):
        _, wbuf, rowsv = bufset
        for j in range(_C // 16):
            wv = wbuf[pl.ds(j * 16, 16)]
            for l in range(16):
                wb = _bcast_lane(wv, l)
                e = j * 16 + l
                for q in range(_H // 16):
                    sl = pl.ds(q * 16, 16)
                    rowsv[e, sl] = rowsv[e, sl] * wb

    def _scatter_start(bufset):
        rowbuf, _, rowsv = bufset
        pltpu.async_copy(rowsv, acc_sh.at[rowbuf], sems, add=True)

    def _scatter_wait(bufset):
        pltpu.make_async_copy(g_hbm.at[pl.ds(0, _C)], bufset[2], sems).wait()

    _issue(0, bufs[0])
    _issue(_C, bufs[1])

    def _ring(i, carry):
        for k in range(3):
            p, q = k, (k + 2) % 3
            _wait_in(bufs[p])
            _scale(bufs[p])
            _scatter_start(bufs[p])
            if k == 0:
                @pl.when(i >= 1)
                def _sw():
                    _scatter_wait(bufs[q])
            else:
                _scatter_wait(bufs[q])
            _issue((i * 3 + k + 2) * _C, bufs[q])
        return carry
    lax.fori_loop(0, _NT, _ring, 0)

    _wait_in(bufs[0])
    _scale(bufs[0])
    _scatter_start(bufs[0])
    _wait_in(bufs[1])
    _scale(bufs[1])
    _scatter_start(bufs[1])
    _scatter_wait(bufs[2])
    _scatter_wait(bufs[0])
    _scatter_wait(bufs[1])

    plsc.subcore_barrier()
    for b in range(_NZ):
        sl = pl.ds(rbase + b * _ZR, _ZR)
        pltpu.async_copy(acc_sh.at[sl], out_hbm.at[c, sl], semi)

    @pl.when(s == 0)
    def _copy_tail():
        sl = pl.ds(_RB * _NS, _TAIL)
        pltpu.async_copy(acc_sh.at[sl], out_hbm.at[c, sl], semi)
    for b in range(_NZ):
        sl = pl.ds(rbase + b * _ZR, _ZR)
        pltpu.make_async_copy(acc_sh.at[sl], out_hbm.at[c, sl], semi).wait()

    @pl.when(s == 0)
    def _copy_tail_wait():
        sl = pl.ds(_RB * _NS, _TAIL)
        pltpu.make_async_copy(acc_sh.at[sl], out_hbm.at[c, sl], semi).wait()


_sc_agg_cached = None


def _sc_agg(*args):
    global _sc_agg_cached
    if _sc_agg_cached is None:
        mesh = plsc.VectorSubcoreMesh(core_axis_name="c", subcore_axis_name="s")
        _sc_agg_cached = pl.kernel(
            _sc_body, mesh=mesh,
            out_type=jax.ShapeDtypeStruct((2, _N, _H), jnp.float32),
            scratch_types=[
                pltpu.VMEM((_EPT,), jnp.int32),
                pltpu.VMEM((_C,), jnp.int32),
                pltpu.VMEM((_C,), jnp.int32),
                pltpu.VMEM((_C,), jnp.int32),
                pltpu.VMEM((_C,), jnp.float32),
                pltpu.VMEM((_C,), jnp.float32),
                pltpu.VMEM((_C,), jnp.float32),
                pltpu.VMEM((_C, _H), jnp.float32),
                pltpu.VMEM((_C, _H), jnp.float32),
                pltpu.VMEM((_C, _H), jnp.float32),
                pltpu.VMEM_SHARED((_N, _H), jnp.float32),
                pltpu.SemaphoreType.DMA,
                pltpu.SemaphoreType.DMA,
                pltpu.SemaphoreType.DMA,
            ])
    return _sc_agg_cached(*args)


def kernel(node_repr, edge_index, edge_weight):
    row = edge_index[0].astype(jnp.int32)
    col = edge_index[1].astype(jnp.int32)
    colp = jnp.concatenate([col, col + _N])
    w = edge_weight.astype(jnp.float32)

    g = _node_first(node_repr.astype(jnp.float32))
    a = _sc_agg(g.reshape(2 * _N, _H), colp, row, w)
    g = _node_mid(a)
    a = _sc_agg(g.reshape(2 * _N, _H), colp, row, w)
    return _node_last(a)
